# baseline (speedup 1.0000x reference)
import jax
import jax.numpy as jnp
from jax import lax
from jax.experimental import pallas as pl
from jax.experimental.pallas import tpu as pltpu
from jax.experimental.pallas import tpu_sc as plsc

N = 10000
D = 128
E = 320000
NCLS = 40
BN_EPS = 1e-5

NC, NS = 2, 16
NW = NC * NS
C = 128
NB = 3
CH = 81
EPT = CH * C
E_PAD = EPT * NW
NCHT = E_PAD // C
NROWS = 10112
ZROWS = NROWS // NS

_mesh = plsc.VectorSubcoreMesh(core_axis_name="c", subcore_axis_name="s",
                               num_cores=NC, num_subcores=NS)


def _sc_agg_body(src2_hbm, dst2_hbm, h_hbm, z_hbm, agg_out,
                 sidx0, sidx1, sidx2, didx0, didx1, didx2,
                 rows0, rows1, rows2, agg_sh,
                 gs0, gs1, gs2, is0, is1, is2, ds0, ds1, ds2,
                 ss0, ss1, ss2):
    rows = (rows0, rows1, rows2)
    sidx = (sidx0, sidx1, sidx2)
    didx = (didx0, didx1, didx2)
    gsem = (gs0, gs1, gs2)
    isem = (is0, is1, is2)
    dsem = (ds0, ds1, ds2)
    ssem = (ss0, ss1, ss2)
    cid = lax.axis_index("c")
    sid = lax.axis_index("s")
    wid = sid * NC + cid
    brow = wid * CH

    pltpu.sync_copy(z_hbm, rows0)
    for j in range(-(-ZROWS // C)):
        n = min(C, ZROWS - j * C)
        pltpu.sync_copy(rows0.at[pl.ds(0, n)],
                        agg_sh.at[pl.ds(sid * ZROWS + j * C, n)])
    plsc.subcore_barrier()

    for b in range(NB):
        pltpu.sync_copy(src2_hbm.at[pl.ds(brow + b, 1)], sidx[b])
        pltpu.sync_copy(dst2_hbm.at[pl.ds(brow + b, 1)], didx[b])
        pltpu.async_copy(h_hbm.at[sidx[b].at[0]], rows[b], gsem[b])

    def gbody(g, carry):
        for b in range(NB):
            c = g * NB + b
            pltpu.make_async_copy(h_hbm.at[sidx[b].at[0]], rows[b],
                                  gsem[b]).wait()
            nxt = jnp.minimum(c + NB, CH - 1)
            pltpu.async_copy(src2_hbm.at[pl.ds(brow + nxt, 1)], sidx[b],
                             isem[b])
            pltpu.async_copy(rows[b], agg_sh.at[didx[b].at[0]], ssem[b],
                             add=True)
        for b in range(NB):
            c = g * NB + b
            nxt = jnp.minimum(c + NB, CH - 1)
            pltpu.make_async_copy(rows[b], agg_sh.at[didx[b].at[0]],
                                  ssem[b]).wait()
            pltpu.async_copy(dst2_hbm.at[pl.ds(brow + nxt, 1)], didx[b],
                             dsem[b])
            pltpu.make_async_copy(src2_hbm.at[pl.ds(brow, 1)], sidx[b],
                                  isem[b]).wait()
            pltpu.make_async_copy(dst2_hbm.at[pl.ds(brow, 1)], didx[b],
                                  dsem[b]).wait()
            pltpu.async_copy(h_hbm.at[sidx[b].at[0]], rows[b], gsem[b])
        return carry
    lax.fori_loop(0, CH // NB, gbody, 0)
    for b in range(NB):
        pltpu.make_async_copy(h_hbm.at[sidx[b].at[0]], rows[b],
                              gsem[b]).wait()
    plsc.subcore_barrier()

    pltpu.sync_copy(agg_sh.at[pl.ds(sid * ZROWS, ZROWS)],
                    agg_out.at[cid, pl.ds(sid * ZROWS, ZROWS)])


_sc_agg = pl.kernel(
    _sc_agg_body,
    out_type=jax.ShapeDtypeStruct((NC, NROWS, D), jnp.float32),
    mesh=_mesh,
    scratch_types=[
        pltpu.VMEM((1, C), jnp.int32),
        pltpu.VMEM((1, C), jnp.int32),
        pltpu.VMEM((1, C), jnp.int32),
        pltpu.VMEM((1, C), jnp.int32),
        pltpu.VMEM((1, C), jnp.int32),
        pltpu.VMEM((1, C), jnp.int32),
        pltpu.VMEM((C, D), jnp.float32),
        pltpu.VMEM((C, D), jnp.float32),
        pltpu.VMEM((C, D), jnp.float32),
        pltpu.VMEM_SHARED((NROWS, D), jnp.float32),
        pltpu.SemaphoreType.DMA,
        pltpu.SemaphoreType.DMA,
        pltpu.SemaphoreType.DMA,
        pltpu.SemaphoreType.DMA,
        pltpu.SemaphoreType.DMA,
        pltpu.SemaphoreType.DMA,
        pltpu.SemaphoreType.DMA,
        pltpu.SemaphoreType.DMA,
        pltpu.SemaphoreType.DMA,
        pltpu.SemaphoreType.DMA,
        pltpu.SemaphoreType.DMA,
        pltpu.SemaphoreType.DMA,
    ])


def _sc_deg_body(dst2_hbm, ones_hbm, z_hbm, deg_out,
                 didx0, didx1, didx2, ones_v, z_v, deg_sh,
                 ds0, ds1, ds2, ss0, ss1, ss2):
    didx = (didx0, didx1, didx2)
    dsem = (ds0, ds1, ds2)
    ssem = (ss0, ss1, ss2)
    cid = lax.axis_index("c")
    sid = lax.axis_index("s")
    wid = sid * NC + cid
    brow = wid * CH

    pltpu.sync_copy(ones_hbm, ones_v)
    pltpu.sync_copy(z_hbm, z_v)
    for j in range(-(-ZROWS // C)):
        n = min(C, ZROWS - j * C)
        pltpu.sync_copy(z_v.at[pl.ds(0, n)],
                        deg_sh.at[pl.ds(sid * ZROWS + j * C, n)])
    plsc.subcore_barrier()

    for b in range(NB):
        pltpu.sync_copy(dst2_hbm.at[pl.ds(brow + b, 1)], didx[b])

    def ebody(g, carry):
        for b in range(NB):
            pltpu.async_copy(ones_v, deg_sh.at[didx[b].at[0]], ssem[b],
                             add=True)
        for b in range(NB):
            c = g * NB + b
            nxt = jnp.minimum(c + NB, CH - 1)
            pltpu.make_async_copy(ones_v, deg_sh.at[didx[b].at[0]],
                                  ssem[b]).wait()
            pltpu.async_copy(dst2_hbm.at[pl.ds(brow + nxt, 1)], didx[b],
                             dsem[b])
            pltpu.make_async_copy(dst2_hbm.at[pl.ds(brow, 1)], didx[b],
                                  dsem[b]).wait()
        return carry
    lax.fori_loop(0, CH // NB, ebody, 0)
    plsc.subcore_barrier()

    pltpu.sync_copy(deg_sh.at[pl.ds(sid * ZROWS, ZROWS)],
                    deg_out.at[cid, pl.ds(sid * ZROWS, ZROWS)])


_sc_deg = pl.kernel(
    _sc_deg_body,
    out_type=jax.ShapeDtypeStruct((NC, NROWS, D), jnp.float32),
    mesh=_mesh,
    scratch_types=[
        pltpu.VMEM((1, C), jnp.int32),
        pltpu.VMEM((1, C), jnp.int32),
        pltpu.VMEM((1, C), jnp.int32),
        pltpu.VMEM((C, D), jnp.float32),
        pltpu.VMEM((C, D), jnp.float32),
        pltpu.VMEM_SHARED((NROWS, D), jnp.float32),
        pltpu.SemaphoreType.DMA,
        pltpu.SemaphoreType.DMA,
        pltpu.SemaphoreType.DMA,
        pltpu.SemaphoreType.DMA,
        pltpu.SemaphoreType.DMA,
        pltpu.SemaphoreType.DMA,
    ])

BR = 1000


def _tc_layer(aggp, degp, h, wcat, scale, shift):
    def body(agg_ref, deg_ref, h_ref, w_ref, sc_ref, sh_ref, o_ref):
        d = deg_ref[0, :, 0:1] + deg_ref[1, :, 0:1]
        mean = (agg_ref[0] + agg_ref[1]) / jnp.maximum(d, 1.0)
        xcat = jnp.concatenate([mean, h_ref[...]], axis=1)
        z = jnp.dot(xcat, w_ref[...], preferred_element_type=jnp.float32)
        o_ref[...] = jnp.maximum(z * sc_ref[...] + sh_ref[...], 0.0)

    return pl.pallas_call(
        body,
        grid=(N // BR,),
        in_specs=[
            pl.BlockSpec((NC, BR, D), lambda i: (0, i, 0)),
            pl.BlockSpec((NC, BR, D), lambda i: (0, i, 0)),
            pl.BlockSpec((BR, D), lambda i: (i, 0)),
            pl.BlockSpec((2 * D, D), lambda i: (0, 0)),
            pl.BlockSpec((1, D), lambda i: (0, 0)),
            pl.BlockSpec((1, D), lambda i: (0, 0)),
        ],
        out_specs=pl.BlockSpec((BR, D), lambda i: (i, 0)),
        out_shape=jax.ShapeDtypeStruct((N, D), jnp.float32),
    )(aggp, degp, h, wcat, scale, shift)


def _tc_layer_head(aggp, degp, h, wcat, scale, shift, wf_t, bf, wc_t, bc):
    def body(agg_ref, deg_ref, h_ref, w_ref, sc_ref, sh_ref,
             wf_ref, bf_ref, wc_ref, bc_ref, o_ref):
        d = deg_ref[0, :, 0:1] + deg_ref[1, :, 0:1]
        mean = (agg_ref[0] + agg_ref[1]) / jnp.maximum(d, 1.0)
        xcat = jnp.concatenate([mean, h_ref[...]], axis=1)
        z = jnp.dot(xcat, w_ref[...], preferred_element_type=jnp.float32)
        h3 = jnp.maximum(z * sc_ref[...] + sh_ref[...], 0.0)
        z = jnp.dot(h3, wf_ref[...], preferred_element_type=jnp.float32)
        z = jnp.maximum(z + bf_ref[...], 0.0)
        o_ref[...] = jnp.dot(z, wc_ref[...],
                             preferred_element_type=jnp.float32) + bc_ref[...]

    return pl.pallas_call(
        body,
        grid=(N // BR,),
        in_specs=[
            pl.BlockSpec((NC, BR, D), lambda i: (0, i, 0)),
            pl.BlockSpec((NC, BR, D), lambda i: (0, i, 0)),
            pl.BlockSpec((BR, D), lambda i: (i, 0)),
            pl.BlockSpec((2 * D, D), lambda i: (0, 0)),
            pl.BlockSpec((1, D), lambda i: (0, 0)),
            pl.BlockSpec((1, D), lambda i: (0, 0)),
            pl.BlockSpec((D, D), lambda i: (0, 0)),
            pl.BlockSpec((1, D), lambda i: (0, 0)),
            pl.BlockSpec((D, NCLS), lambda i: (0, 0)),
            pl.BlockSpec((1, NCLS), lambda i: (0, 0)),
        ],
        out_specs=pl.BlockSpec((BR, NCLS), lambda i: (i, 0)),
        out_shape=jax.ShapeDtypeStruct((N, NCLS), jnp.float32),
    )(aggp, degp, h, wcat, scale, shift, wf_t, bf, wc_t, bc)


def kernel(x, edge_index, W1l, b1, W1r, g1, be1, W2l, b2, W2r, g2, be2,
           W3l, b3, W3r, g3, be3, Wf, bf, Wc, bc):
    f32 = jnp.float32
    src = edge_index[0]
    dst = edge_index[1]
    pad = jnp.arange(E_PAD - E, dtype=jnp.int32)
    src_p = jnp.concatenate([src, pad % 1024]).reshape(NCHT, C)
    dst_p = jnp.concatenate([dst, N + pad % (NROWS - N)]).reshape(NCHT, C)
    zst = jnp.zeros((C, D), f32)
    ones_c = jnp.ones((C, D), f32)

    inv_s = (1.0 / jnp.sqrt(jnp.asarray(1.0 + BN_EPS, f32))).astype(f32)

    def mk(Wl, bl, Wr, g, be):
        wcat = jnp.concatenate([Wl.T, Wr.T], axis=0)
        scale = (g * inv_s)[None, :]
        shift = (bl * g * inv_s + be)[None, :]
        return wcat, scale, shift

    w1 = mk(W1l, b1, W1r, g1, be1)
    w2 = mk(W2l, b2, W2r, g2, be2)
    w3 = mk(W3l, b3, W3r, g3, be3)

    degp = _sc_deg(dst_p, ones_c, zst)
    aggp = _sc_agg(src_p, dst_p, x, zst)
    h = _tc_layer(aggp, degp, x, *w1)
    aggp = _sc_agg(src_p, dst_p, h, zst)
    h = _tc_layer(aggp, degp, h, *w2)
    aggp = _sc_agg(src_p, dst_p, h, zst)
    return _tc_layer_head(aggp, degp, h, *w3,
                          Wf.T, bf[None, :], Wc.T, bc[None, :])


# speedup vs baseline: 1.0394x; 1.0394x over previous
import jax
import jax.numpy as jnp
from jax import lax
from jax.experimental import pallas as pl
from jax.experimental.pallas import tpu as pltpu
from jax.experimental.pallas import tpu_sc as plsc

N = 10000
D = 128
E = 320000
NCLS = 40
BN_EPS = 1e-5

NC, NS = 2, 16
NW = NC * NS
C = 88
NB = 3
CH = 114
EPT = CH * C
E_PAD = EPT * NW
NROWS = 10112
ZROWS = NROWS // NS

_mesh = plsc.VectorSubcoreMesh(core_axis_name="c", subcore_axis_name="s",
                               num_cores=NC, num_subcores=NS)


def _sc_agg_body(src2_hbm, dst3_hbm, h_hbm, z_hbm, agg_out,
                 sidx_v, dst2_v, rows0, rows1, rows2, agg_sh,
                 gs0, gs1, gs2, is0, is1, is2, ss0, ss1, ss2):
    rows = (rows0, rows1, rows2)
    gsem = (gs0, gs1, gs2)
    isem = (is0, is1, is2)
    ssem = (ss0, ss1, ss2)
    cid = lax.axis_index("c")
    sid = lax.axis_index("s")
    wid = sid * NC + cid
    brow = wid * CH

    pltpu.sync_copy(dst3_hbm.at[wid], dst2_v)
    pltpu.sync_copy(z_hbm, rows0)
    for j in range(-(-ZROWS // C)):
        n = min(C, ZROWS - j * C)
        pltpu.sync_copy(rows0.at[pl.ds(0, n)],
                        agg_sh.at[pl.ds(sid * ZROWS + j * C, n)])
    plsc.subcore_barrier()

    for b in range(NB):
        pltpu.sync_copy(src2_hbm.at[pl.ds(brow + b, 1)],
                        sidx_v.at[pl.ds(b, 1)])
        pltpu.async_copy(h_hbm.at[sidx_v.at[b]], rows[b], gsem[b])

    def gbody(g, carry):
        for b in range(NB):
            c = g * NB + b
            pltpu.make_async_copy(h_hbm.at[sidx_v.at[b]], rows[b],
                                  gsem[b]).wait()
            pltpu.async_copy(src2_hbm.at[pl.ds(brow + c + NB, 1)],
                             sidx_v.at[pl.ds(b, 1)], isem[b])
            pltpu.async_copy(rows[b], agg_sh.at[dst2_v.at[c]], ssem[b],
                             add=True)
        for b in range(NB):
            c = g * NB + b
            pltpu.make_async_copy(rows[b], agg_sh.at[dst2_v.at[c]],
                                  ssem[b]).wait()
            pltpu.make_async_copy(src2_hbm.at[pl.ds(brow, 1)],
                                  sidx_v.at[pl.ds(b, 1)], isem[b]).wait()
            pltpu.async_copy(h_hbm.at[sidx_v.at[b]], rows[b], gsem[b])
        return carry
    lax.fori_loop(0, CH // NB - 1, gbody, 0)
    for b in range(NB):
        c = CH - NB + b
        pltpu.make_async_copy(h_hbm.at[sidx_v.at[b]], rows[b],
                              gsem[b]).wait()
        pltpu.async_copy(rows[b], agg_sh.at[dst2_v.at[c]], ssem[b],
                         add=True)
    for b in range(NB):
        c = CH - NB + b
        pltpu.make_async_copy(rows[b], agg_sh.at[dst2_v.at[c]],
                              ssem[b]).wait()
    plsc.subcore_barrier()

    pltpu.sync_copy(agg_sh.at[pl.ds(sid * ZROWS, ZROWS)],
                    agg_out.at[cid, pl.ds(sid * ZROWS, ZROWS)])


_sc_agg = pl.kernel(
    _sc_agg_body,
    out_type=jax.ShapeDtypeStruct((NC, NROWS, D), jnp.float32),
    mesh=_mesh,
    scratch_types=[
        pltpu.VMEM((NB, C), jnp.int32),
        pltpu.VMEM((CH, C), jnp.int32),
        pltpu.VMEM((C, D), jnp.float32),
        pltpu.VMEM((C, D), jnp.float32),
        pltpu.VMEM((C, D), jnp.float32),
        pltpu.VMEM_SHARED((NROWS, D), jnp.float32),
        pltpu.SemaphoreType.DMA,
        pltpu.SemaphoreType.DMA,
        pltpu.SemaphoreType.DMA,
        pltpu.SemaphoreType.DMA,
        pltpu.SemaphoreType.DMA,
        pltpu.SemaphoreType.DMA,
        pltpu.SemaphoreType.DMA,
        pltpu.SemaphoreType.DMA,
        pltpu.SemaphoreType.DMA,
    ])


def _sc_deg_body(dst3_hbm, ones_hbm, z_hbm, deg_out,
                 dst2_v, ones_v, z_v, deg_sh, ss0, ss1, ss2):
    ssem = (ss0, ss1, ss2)
    cid = lax.axis_index("c")
    sid = lax.axis_index("s")
    wid = sid * NC + cid

    pltpu.sync_copy(dst3_hbm.at[wid], dst2_v)
    pltpu.sync_copy(ones_hbm, ones_v)
    pltpu.sync_copy(z_hbm, z_v)
    for j in range(-(-ZROWS // C)):
        n = min(C, ZROWS - j * C)
        pltpu.sync_copy(z_v.at[pl.ds(0, n)],
                        deg_sh.at[pl.ds(sid * ZROWS + j * C, n)])
    plsc.subcore_barrier()

    for b in range(NB):
        pltpu.async_copy(ones_v, deg_sh.at[dst2_v.at[b]], ssem[b], add=True)

    def ebody(g, carry):
        for b in range(NB):
            c = g * NB + b
            pltpu.make_async_copy(ones_v, deg_sh.at[dst2_v.at[c]],
                                  ssem[b]).wait()
            pltpu.async_copy(ones_v, deg_sh.at[dst2_v.at[c + NB]], ssem[b],
                             add=True)
        return carry
    lax.fori_loop(0, CH // NB - 1, ebody, 0)
    for b in range(NB):
        pltpu.make_async_copy(ones_v, deg_sh.at[dst2_v.at[0]],
                              ssem[b]).wait()
    plsc.subcore_barrier()

    pltpu.sync_copy(deg_sh.at[pl.ds(sid * ZROWS, ZROWS)],
                    deg_out.at[cid, pl.ds(sid * ZROWS, ZROWS)])


_sc_deg = pl.kernel(
    _sc_deg_body,
    out_type=jax.ShapeDtypeStruct((NC, NROWS, D), jnp.float32),
    mesh=_mesh,
    scratch_types=[
        pltpu.VMEM((CH, C), jnp.int32),
        pltpu.VMEM((C, D), jnp.float32),
        pltpu.VMEM((C, D), jnp.float32),
        pltpu.VMEM_SHARED((NROWS, D), jnp.float32),
        pltpu.SemaphoreType.DMA,
        pltpu.SemaphoreType.DMA,
        pltpu.SemaphoreType.DMA,
    ])

BR = 1000


def _tc_layer(aggp, degp, h, wcat, scale, shift):
    def body(agg_ref, deg_ref, h_ref, w_ref, sc_ref, sh_ref, o_ref):
        d = deg_ref[0, :, 0:1] + deg_ref[1, :, 0:1]
        mean = (agg_ref[0] + agg_ref[1]) / jnp.maximum(d, 1.0)
        xcat = jnp.concatenate([mean, h_ref[...]], axis=1)
        z = jnp.dot(xcat, w_ref[...], preferred_element_type=jnp.float32)
        o_ref[...] = jnp.maximum(z * sc_ref[...] + sh_ref[...], 0.0)

    return pl.pallas_call(
        body,
        grid=(N // BR,),
        in_specs=[
            pl.BlockSpec((NC, BR, D), lambda i: (0, i, 0)),
            pl.BlockSpec((NC, BR, D), lambda i: (0, i, 0)),
            pl.BlockSpec((BR, D), lambda i: (i, 0)),
            pl.BlockSpec((2 * D, D), lambda i: (0, 0)),
            pl.BlockSpec((1, D), lambda i: (0, 0)),
            pl.BlockSpec((1, D), lambda i: (0, 0)),
        ],
        out_specs=pl.BlockSpec((BR, D), lambda i: (i, 0)),
        out_shape=jax.ShapeDtypeStruct((N, D), jnp.float32),
    )(aggp, degp, h, wcat, scale, shift)


def _tc_layer_head(aggp, degp, h, wcat, scale, shift, wf_t, bf, wc_t, bc):
    def body(agg_ref, deg_ref, h_ref, w_ref, sc_ref, sh_ref,
             wf_ref, bf_ref, wc_ref, bc_ref, o_ref):
        d = deg_ref[0, :, 0:1] + deg_ref[1, :, 0:1]
        mean = (agg_ref[0] + agg_ref[1]) / jnp.maximum(d, 1.0)
        xcat = jnp.concatenate([mean, h_ref[...]], axis=1)
        z = jnp.dot(xcat, w_ref[...], preferred_element_type=jnp.float32)
        h3 = jnp.maximum(z * sc_ref[...] + sh_ref[...], 0.0)
        z = jnp.dot(h3, wf_ref[...], preferred_element_type=jnp.float32)
        z = jnp.maximum(z + bf_ref[...], 0.0)
        o_ref[...] = jnp.dot(z, wc_ref[...],
                             preferred_element_type=jnp.float32) + bc_ref[...]

    return pl.pallas_call(
        body,
        grid=(N // BR,),
        in_specs=[
            pl.BlockSpec((NC, BR, D), lambda i: (0, i, 0)),
            pl.BlockSpec((NC, BR, D), lambda i: (0, i, 0)),
            pl.BlockSpec((BR, D), lambda i: (i, 0)),
            pl.BlockSpec((2 * D, D), lambda i: (0, 0)),
            pl.BlockSpec((1, D), lambda i: (0, 0)),
            pl.BlockSpec((1, D), lambda i: (0, 0)),
            pl.BlockSpec((D, D), lambda i: (0, 0)),
            pl.BlockSpec((1, D), lambda i: (0, 0)),
            pl.BlockSpec((D, NCLS), lambda i: (0, 0)),
            pl.BlockSpec((1, NCLS), lambda i: (0, 0)),
        ],
        out_specs=pl.BlockSpec((BR, NCLS), lambda i: (i, 0)),
        out_shape=jax.ShapeDtypeStruct((N, NCLS), jnp.float32),
    )(aggp, degp, h, wcat, scale, shift, wf_t, bf, wc_t, bc)


def kernel(x, edge_index, W1l, b1, W1r, g1, be1, W2l, b2, W2r, g2, be2,
           W3l, b3, W3r, g3, be3, Wf, bf, Wc, bc):
    f32 = jnp.float32
    src = edge_index[0]
    dst = edge_index[1]
    pad = jnp.arange(E_PAD - E, dtype=jnp.int32)
    src_p = jnp.concatenate([src, pad % 1024]).reshape(NW * CH, C)
    dst_p = jnp.concatenate([dst, N + pad % (NROWS - N)]).reshape(NW, CH, C)
    zst = jnp.zeros((C, D), f32)
    ones_c = jnp.ones((C, D), f32)

    inv_s = (1.0 / jnp.sqrt(jnp.asarray(1.0 + BN_EPS, f32))).astype(f32)

    def mk(Wl, bl, Wr, g, be):
        wcat = jnp.concatenate([Wl.T, Wr.T], axis=0)
        scale = (g * inv_s)[None, :]
        shift = (bl * g * inv_s + be)[None, :]
        return wcat, scale, shift

    w1 = mk(W1l, b1, W1r, g1, be1)
    w2 = mk(W2l, b2, W2r, g2, be2)
    w3 = mk(W3l, b3, W3r, g3, be3)

    degp = _sc_deg(dst_p, ones_c, zst)
    aggp = _sc_agg(src_p, dst_p, x, zst)
    h = _tc_layer(aggp, degp, x, *w1)
    aggp = _sc_agg(src_p, dst_p, h, zst)
    h = _tc_layer(aggp, degp, h, *w2)
    aggp = _sc_agg(src_p, dst_p, h, zst)
    return _tc_layer_head(aggp, degp, h, *w3,
                          Wf.T, bf[None, :], Wc.T, bc[None, :])


# speedup vs baseline: 1.1271x; 1.0844x over previous
import jax
import jax.numpy as jnp
from jax import lax
from jax.experimental import pallas as pl
from jax.experimental.pallas import tpu as pltpu
from jax.experimental.pallas import tpu_sc as plsc

N = 10000
D = 128
E = 320000
NCLS = 40
BN_EPS = 1e-5

NC, NS = 2, 16
NW = NC * NS
C = 128
NB = 2
CH = 80
EPT = CH * C
E_PAD = EPT * NW
NROWS = 10240
ZROWS = NROWS // NS

_mesh = plsc.VectorSubcoreMesh(core_axis_name="c", subcore_axis_name="s",
                               num_cores=NC, num_subcores=NS)


def _sc_agg_body(src2_hbm, dst2_hbm, h_hbm, z_hbm, agg_out,
                 dst2_v, sidx0, sidx1, rows0, rows1, agg_sh,
                 gs0, gs1, is0, is1, ss0, ss1):
    rows = (rows0, rows1)
    sidx = (sidx0, sidx1)
    gsem = (gs0, gs1)
    isem = (is0, is1)
    ssem = (ss0, ss1)
    cid = lax.axis_index("c")
    sid = lax.axis_index("s")
    wid = sid * NC + cid
    brow = wid * CH

    pltpu.sync_copy(dst2_hbm.at[pl.ds(brow, CH)], dst2_v)
    pltpu.sync_copy(z_hbm, rows0)
    for j in range(ZROWS // C):
        pltpu.sync_copy(rows0, agg_sh.at[pl.ds(sid * ZROWS + j * C, C)])
    plsc.subcore_barrier()

    for b in range(NB):
        pltpu.sync_copy(src2_hbm.at[pl.ds(brow + b, 1)], sidx[b])
        pltpu.async_copy(h_hbm.at[sidx[b].at[0]], rows[b], gsem[b])

    def gbody(g, carry):
        for b in range(NB):
            c = g * NB + b
            pltpu.make_async_copy(h_hbm.at[sidx[b].at[0]], rows[b],
                                  gsem[b]).wait()
            pltpu.async_copy(src2_hbm.at[pl.ds(brow + c + NB, 1)], sidx[b],
                             isem[b])
            pltpu.async_copy(rows[b], agg_sh.at[dst2_v.at[c]], ssem[b],
                             add=True)
            pltpu.make_async_copy(rows[b], agg_sh.at[dst2_v.at[c]],
                                  ssem[b]).wait()
            pltpu.make_async_copy(src2_hbm.at[pl.ds(brow, 1)], sidx[b],
                                  isem[b]).wait()
            pltpu.async_copy(h_hbm.at[sidx[b].at[0]], rows[b], gsem[b])
        return carry
    lax.fori_loop(0, CH // NB - 1, gbody, 0)
    for b in range(NB):
        c = CH - NB + b
        pltpu.make_async_copy(h_hbm.at[sidx[b].at[0]], rows[b],
                              gsem[b]).wait()
        pltpu.async_copy(rows[b], agg_sh.at[dst2_v.at[c]], ssem[b],
                         add=True)
    for b in range(NB):
        c = CH - NB + b
        pltpu.make_async_copy(rows[b], agg_sh.at[dst2_v.at[c]],
                              ssem[b]).wait()
    plsc.subcore_barrier()

    pltpu.sync_copy(agg_sh.at[pl.ds(sid * ZROWS, ZROWS)],
                    agg_out.at[cid, pl.ds(sid * ZROWS, ZROWS)])


_sc_agg = pl.kernel(
    _sc_agg_body,
    out_type=jax.ShapeDtypeStruct((NC, NROWS, D), jnp.float32),
    mesh=_mesh,
    scratch_types=[
        pltpu.VMEM((CH, C), jnp.int32),
        pltpu.VMEM((1, C), jnp.int32),
        pltpu.VMEM((1, C), jnp.int32),
        pltpu.VMEM((C, D), jnp.float32),
        pltpu.VMEM((C, D), jnp.float32),
        pltpu.VMEM_SHARED((NROWS, D), jnp.float32),
        pltpu.SemaphoreType.DMA,
        pltpu.SemaphoreType.DMA,
        pltpu.SemaphoreType.DMA,
        pltpu.SemaphoreType.DMA,
        pltpu.SemaphoreType.DMA,
        pltpu.SemaphoreType.DMA,
    ])


def _sc_deg_body(dst2_hbm, ones_hbm, z_hbm, deg_out,
                 dst2_v, ones_v, z_v, deg_sh, ss0, ss1):
    ssem = (ss0, ss1)
    NQ = 2
    cid = lax.axis_index("c")
    sid = lax.axis_index("s")
    wid = sid * NC + cid
    brow = wid * CH

    pltpu.sync_copy(dst2_hbm.at[pl.ds(brow, CH)], dst2_v)
    pltpu.sync_copy(ones_hbm, ones_v)
    pltpu.sync_copy(z_hbm, z_v)
    for j in range(ZROWS // C):
        pltpu.sync_copy(z_v, deg_sh.at[pl.ds(sid * ZROWS + j * C, C)])
    plsc.subcore_barrier()

    for b in range(NQ):
        pltpu.async_copy(ones_v, deg_sh.at[dst2_v.at[b]], ssem[b], add=True)

    def ebody(g, carry):
        for b in range(NQ):
            c = g * NQ + b
            pltpu.make_async_copy(ones_v, deg_sh.at[dst2_v.at[c]],
                                  ssem[b]).wait()
            pltpu.async_copy(ones_v, deg_sh.at[dst2_v.at[c + NQ]], ssem[b],
                             add=True)
        return carry
    lax.fori_loop(0, CH // NQ - 1, ebody, 0)
    for b in range(NQ):
        pltpu.make_async_copy(ones_v, deg_sh.at[dst2_v.at[0]],
                              ssem[b]).wait()
    plsc.subcore_barrier()

    pltpu.sync_copy(deg_sh.at[pl.ds(sid * ZROWS, ZROWS)],
                    deg_out.at[cid, pl.ds(sid * ZROWS, ZROWS)])


_sc_deg = pl.kernel(
    _sc_deg_body,
    out_type=jax.ShapeDtypeStruct((NC, NROWS, D), jnp.float32),
    mesh=_mesh,
    scratch_types=[
        pltpu.VMEM((CH, C), jnp.int32),
        pltpu.VMEM((C, D), jnp.float32),
        pltpu.VMEM((C, D), jnp.float32),
        pltpu.VMEM_SHARED((NROWS, D), jnp.float32),
        pltpu.SemaphoreType.DMA,
        pltpu.SemaphoreType.DMA,
    ])

BR = 1000


def _tc_layer(aggp, degp, h, wcat, scale, shift):
    def body(agg_ref, deg_ref, h_ref, w_ref, sc_ref, sh_ref, o_ref):
        d = deg_ref[0, :, 0:1] + deg_ref[1, :, 0:1]
        mean = (agg_ref[0] + agg_ref[1]) / jnp.maximum(d, 1.0)
        xcat = jnp.concatenate([mean, h_ref[...]], axis=1)
        z = jnp.dot(xcat, w_ref[...], preferred_element_type=jnp.float32)
        o_ref[...] = jnp.maximum(z * sc_ref[...] + sh_ref[...], 0.0)

    return pl.pallas_call(
        body,
        grid=(N // BR,),
        in_specs=[
            pl.BlockSpec((NC, BR, D), lambda i: (0, i, 0)),
            pl.BlockSpec((NC, BR, D), lambda i: (0, i, 0)),
            pl.BlockSpec((BR, D), lambda i: (i, 0)),
            pl.BlockSpec((2 * D, D), lambda i: (0, 0)),
            pl.BlockSpec((1, D), lambda i: (0, 0)),
            pl.BlockSpec((1, D), lambda i: (0, 0)),
        ],
        out_specs=pl.BlockSpec((BR, D), lambda i: (i, 0)),
        out_shape=jax.ShapeDtypeStruct((N, D), jnp.float32),
    )(aggp, degp, h, wcat, scale, shift)


def _tc_layer_head(aggp, degp, h, wcat, scale, shift, wf_t, bf, wc_t, bc):
    def body(agg_ref, deg_ref, h_ref, w_ref, sc_ref, sh_ref,
             wf_ref, bf_ref, wc_ref, bc_ref, o_ref):
        d = deg_ref[0, :, 0:1] + deg_ref[1, :, 0:1]
        mean = (agg_ref[0] + agg_ref[1]) / jnp.maximum(d, 1.0)
        xcat = jnp.concatenate([mean, h_ref[...]], axis=1)
        z = jnp.dot(xcat, w_ref[...], preferred_element_type=jnp.float32)
        h3 = jnp.maximum(z * sc_ref[...] + sh_ref[...], 0.0)
        z = jnp.dot(h3, wf_ref[...], preferred_element_type=jnp.float32)
        z = jnp.maximum(z + bf_ref[...], 0.0)
        o_ref[...] = jnp.dot(z, wc_ref[...],
                             preferred_element_type=jnp.float32) + bc_ref[...]

    return pl.pallas_call(
        body,
        grid=(N // BR,),
        in_specs=[
            pl.BlockSpec((NC, BR, D), lambda i: (0, i, 0)),
            pl.BlockSpec((NC, BR, D), lambda i: (0, i, 0)),
            pl.BlockSpec((BR, D), lambda i: (i, 0)),
            pl.BlockSpec((2 * D, D), lambda i: (0, 0)),
            pl.BlockSpec((1, D), lambda i: (0, 0)),
            pl.BlockSpec((1, D), lambda i: (0, 0)),
            pl.BlockSpec((D, D), lambda i: (0, 0)),
            pl.BlockSpec((1, D), lambda i: (0, 0)),
            pl.BlockSpec((D, NCLS), lambda i: (0, 0)),
            pl.BlockSpec((1, NCLS), lambda i: (0, 0)),
        ],
        out_specs=pl.BlockSpec((BR, NCLS), lambda i: (i, 0)),
        out_shape=jax.ShapeDtypeStruct((N, NCLS), jnp.float32),
    )(aggp, degp, h, wcat, scale, shift, wf_t, bf, wc_t, bc)


def kernel(x, edge_index, W1l, b1, W1r, g1, be1, W2l, b2, W2r, g2, be2,
           W3l, b3, W3r, g3, be3, Wf, bf, Wc, bc):
    f32 = jnp.float32
    src = edge_index[0]
    dst = edge_index[1]
    pad = jnp.arange(E_PAD - E, dtype=jnp.int32)
    src_p = jnp.concatenate([src, pad % 1024]).reshape(NW * CH, C)
    dst_p = jnp.concatenate([dst, N + pad % (NROWS - N)]).reshape(NW * CH, C)
    zst = jnp.zeros((C, D), f32)
    ones_c = jnp.ones((C, D), f32)

    inv_s = (1.0 / jnp.sqrt(jnp.asarray(1.0 + BN_EPS, f32))).astype(f32)

    def mk(Wl, bl, Wr, g, be):
        wcat = jnp.concatenate([Wl.T, Wr.T], axis=0)
        scale = (g * inv_s)[None, :]
        shift = (bl * g * inv_s + be)[None, :]
        return wcat, scale, shift

    w1 = mk(W1l, b1, W1r, g1, be1)
    w2 = mk(W2l, b2, W2r, g2, be2)
    w3 = mk(W3l, b3, W3r, g3, be3)

    degp = _sc_deg(dst_p, ones_c, zst)
    aggp = _sc_agg(src_p, dst_p, x, zst)
    h = _tc_layer(aggp, degp, x, *w1)
    aggp = _sc_agg(src_p, dst_p, h, zst)
    h = _tc_layer(aggp, degp, h, *w2)
    aggp = _sc_agg(src_p, dst_p, h, zst)
    return _tc_layer_head(aggp, degp, h, *w3,
                          Wf.T, bf[None, :], Wc.T, bc[None, :])


# speedup vs baseline: 1.1281x; 1.0009x over previous
import jax
import jax.numpy as jnp
from jax import lax
from jax.experimental import pallas as pl
from jax.experimental.pallas import tpu as pltpu
from jax.experimental.pallas import tpu_sc as plsc

N = 10000
D = 128
E = 320000
NCLS = 40
BN_EPS = 1e-5

NC, NS = 2, 16
NW = NC * NS
C = 128
NB = 2
CH = 80
EPT = CH * C
E_PAD = EPT * NW
NROWS = 10240
ZROWS = NROWS // NS

_mesh = plsc.VectorSubcoreMesh(core_axis_name="c", subcore_axis_name="s",
                               num_cores=NC, num_subcores=NS)


def _sc_agg_body(src2_hbm, dst2_hbm, h_hbm, z_hbm, agg_out,
                 dst2_v, sidx0, sidx1, rows0, rows1, agg_sh,
                 gs0, gs1, is0, is1, ss0, ss1):
    rows = (rows0, rows1)
    sidx = (sidx0, sidx1)
    gsem = (gs0, gs1)
    isem = (is0, is1)
    ssem = (ss0, ss1)
    cid = lax.axis_index("c")
    sid = lax.axis_index("s")
    wid = sid * NC + cid
    brow = wid * CH

    pltpu.sync_copy(dst2_hbm.at[pl.ds(brow, CH)], dst2_v)
    pltpu.sync_copy(z_hbm, rows0)
    for j in range(ZROWS // C):
        pltpu.sync_copy(rows0, agg_sh.at[pl.ds(sid * ZROWS + j * C, C)])
    plsc.subcore_barrier()

    for b in range(NB):
        pltpu.sync_copy(src2_hbm.at[pl.ds(brow + b, 1)], sidx[b])
        pltpu.async_copy(h_hbm.at[sidx[b].at[0]], rows[b], gsem[b])

    def gbody(g, carry):
        for b in range(NB):
            c = g * NB + b
            pltpu.make_async_copy(h_hbm.at[sidx[b].at[0]], rows[b],
                                  gsem[b]).wait()
            pltpu.async_copy(src2_hbm.at[pl.ds(brow + c + NB, 1)], sidx[b],
                             isem[b])
            pltpu.async_copy(rows[b], agg_sh.at[dst2_v.at[c]], ssem[b],
                             add=True)
            pltpu.make_async_copy(rows[b], agg_sh.at[dst2_v.at[c]],
                                  ssem[b]).wait()
            pltpu.make_async_copy(src2_hbm.at[pl.ds(brow, 1)], sidx[b],
                                  isem[b]).wait()
            pltpu.async_copy(h_hbm.at[sidx[b].at[0]], rows[b], gsem[b])
        return carry
    lax.fori_loop(0, CH // NB - 1, gbody, 0)
    for b in range(NB):
        c = CH - NB + b
        pltpu.make_async_copy(h_hbm.at[sidx[b].at[0]], rows[b],
                              gsem[b]).wait()
        pltpu.async_copy(rows[b], agg_sh.at[dst2_v.at[c]], ssem[b],
                         add=True)
    for b in range(NB):
        c = CH - NB + b
        pltpu.make_async_copy(rows[b], agg_sh.at[dst2_v.at[c]],
                              ssem[b]).wait()
    plsc.subcore_barrier()

    pltpu.sync_copy(agg_sh.at[pl.ds(sid * ZROWS, ZROWS)],
                    agg_out.at[cid, pl.ds(sid * ZROWS, ZROWS)])


_sc_agg = pl.kernel(
    _sc_agg_body,
    out_type=jax.ShapeDtypeStruct((NC, NROWS, D), jnp.float32),
    mesh=_mesh,
    scratch_types=[
        pltpu.VMEM((CH, C), jnp.int32),
        pltpu.VMEM((1, C), jnp.int32),
        pltpu.VMEM((1, C), jnp.int32),
        pltpu.VMEM((C, D), jnp.float32),
        pltpu.VMEM((C, D), jnp.float32),
        pltpu.VMEM_SHARED((NROWS, D), jnp.float32),
        pltpu.SemaphoreType.DMA,
        pltpu.SemaphoreType.DMA,
        pltpu.SemaphoreType.DMA,
        pltpu.SemaphoreType.DMA,
        pltpu.SemaphoreType.DMA,
        pltpu.SemaphoreType.DMA,
    ])


def _sc_deg_body(dst2_hbm, ones_hbm, z_hbm, deg_out,
                 dst2_v, ones_v, z_v, deg_sh, ss0, ss1):
    ssem = (ss0, ss1)
    NQ = 2
    cid = lax.axis_index("c")
    sid = lax.axis_index("s")
    wid = sid * NC + cid
    brow = wid * CH

    pltpu.sync_copy(dst2_hbm.at[pl.ds(brow, CH)], dst2_v)
    pltpu.sync_copy(ones_hbm, ones_v)
    pltpu.sync_copy(z_hbm, z_v)
    for j in range(ZROWS // C):
        pltpu.sync_copy(z_v, deg_sh.at[pl.ds(sid * ZROWS + j * C, C)])
    plsc.subcore_barrier()

    for b in range(NQ):
        pltpu.async_copy(ones_v, deg_sh.at[dst2_v.at[b]], ssem[b], add=True)

    def ebody(g, carry):
        for b in range(NQ):
            c = g * NQ + b
            pltpu.make_async_copy(ones_v, deg_sh.at[dst2_v.at[c]],
                                  ssem[b]).wait()
            pltpu.async_copy(ones_v, deg_sh.at[dst2_v.at[c + NQ]], ssem[b],
                             add=True)
        return carry
    lax.fori_loop(0, CH // NQ - 1, ebody, 0)
    for b in range(NQ):
        pltpu.make_async_copy(ones_v, deg_sh.at[dst2_v.at[0]],
                              ssem[b]).wait()
    plsc.subcore_barrier()

    pltpu.sync_copy(deg_sh.at[pl.ds(sid * ZROWS, ZROWS)],
                    deg_out.at[cid, pl.ds(sid * ZROWS, ZROWS)])


_sc_deg = pl.kernel(
    _sc_deg_body,
    out_type=jax.ShapeDtypeStruct((NC, NROWS, D), jnp.float32),
    mesh=_mesh,
    scratch_types=[
        pltpu.VMEM((CH, C), jnp.int32),
        pltpu.VMEM((C, D), jnp.float32),
        pltpu.VMEM((C, D), jnp.float32),
        pltpu.VMEM_SHARED((NROWS, D), jnp.float32),
        pltpu.SemaphoreType.DMA,
        pltpu.SemaphoreType.DMA,
    ])

BR = 1000


def _tc_layer(aggp, degp, h, wcat, scale, shift):
    def body(agg_ref, deg_ref, h_ref, w_ref, sc_ref, sh_ref, o_ref):
        d = (deg_ref[0, :, 0:1] + deg_ref[1, :, 0:1]).astype(jnp.float32)
        mean = (agg_ref[0] + agg_ref[1]) / jnp.maximum(d, 1.0)
        xcat = jnp.concatenate([mean, h_ref[...]], axis=1)
        z = jnp.dot(xcat, w_ref[...], preferred_element_type=jnp.float32)
        o_ref[...] = jnp.maximum(z * sc_ref[...] + sh_ref[...], 0.0)

    return pl.pallas_call(
        body,
        grid=(N // BR,),
        in_specs=[
            pl.BlockSpec((NC, BR, D), lambda i: (0, i, 0)),
            pl.BlockSpec((NC, BR, D), lambda i: (0, i, 0)),
            pl.BlockSpec((BR, D), lambda i: (i, 0)),
            pl.BlockSpec((2 * D, D), lambda i: (0, 0)),
            pl.BlockSpec((1, D), lambda i: (0, 0)),
            pl.BlockSpec((1, D), lambda i: (0, 0)),
        ],
        out_specs=pl.BlockSpec((BR, D), lambda i: (i, 0)),
        out_shape=jax.ShapeDtypeStruct((N, D), jnp.float32),
    )(aggp, degp, h, wcat, scale, shift)


def _tc_layer_head(aggp, degp, h, wcat, scale, shift, wf_t, bf, wc_t, bc):
    def body(agg_ref, deg_ref, h_ref, w_ref, sc_ref, sh_ref,
             wf_ref, bf_ref, wc_ref, bc_ref, o_ref):
        d = (deg_ref[0, :, 0:1] + deg_ref[1, :, 0:1]).astype(jnp.float32)
        mean = (agg_ref[0] + agg_ref[1]) / jnp.maximum(d, 1.0)
        xcat = jnp.concatenate([mean, h_ref[...]], axis=1)
        z = jnp.dot(xcat, w_ref[...], preferred_element_type=jnp.float32)
        h3 = jnp.maximum(z * sc_ref[...] + sh_ref[...], 0.0)
        z = jnp.dot(h3, wf_ref[...], preferred_element_type=jnp.float32)
        z = jnp.maximum(z + bf_ref[...], 0.0)
        o_ref[...] = jnp.dot(z, wc_ref[...],
                             preferred_element_type=jnp.float32) + bc_ref[...]

    return pl.pallas_call(
        body,
        grid=(N // BR,),
        in_specs=[
            pl.BlockSpec((NC, BR, D), lambda i: (0, i, 0)),
            pl.BlockSpec((NC, BR, D), lambda i: (0, i, 0)),
            pl.BlockSpec((BR, D), lambda i: (i, 0)),
            pl.BlockSpec((2 * D, D), lambda i: (0, 0)),
            pl.BlockSpec((1, D), lambda i: (0, 0)),
            pl.BlockSpec((1, D), lambda i: (0, 0)),
            pl.BlockSpec((D, D), lambda i: (0, 0)),
            pl.BlockSpec((1, D), lambda i: (0, 0)),
            pl.BlockSpec((D, NCLS), lambda i: (0, 0)),
            pl.BlockSpec((1, NCLS), lambda i: (0, 0)),
        ],
        out_specs=pl.BlockSpec((BR, NCLS), lambda i: (i, 0)),
        out_shape=jax.ShapeDtypeStruct((N, NCLS), jnp.float32),
    )(aggp, degp, h, wcat, scale, shift, wf_t, bf, wc_t, bc)


def kernel(x, edge_index, W1l, b1, W1r, g1, be1, W2l, b2, W2r, g2, be2,
           W3l, b3, W3r, g3, be3, Wf, bf, Wc, bc):
    f32 = jnp.float32
    src = edge_index[0]
    dst = edge_index[1]
    pad = jnp.arange(E_PAD - E, dtype=jnp.int32)
    src_p = jnp.concatenate([src, pad % 1024]).reshape(NW * CH, C)
    dst_p = jnp.concatenate([dst, N + pad % (NROWS - N)]).reshape(NW * CH, C)
    zst = jnp.zeros((C, D), f32)
    ones_c = jnp.ones((C, D), f32)

    inv_s = (1.0 / jnp.sqrt(jnp.asarray(1.0 + BN_EPS, f32))).astype(f32)

    def mk(Wl, bl, Wr, g, be):
        wcat = jnp.concatenate([Wl.T, Wr.T], axis=0)
        scale = (g * inv_s)[None, :]
        shift = (bl * g * inv_s + be)[None, :]
        return wcat, scale, shift

    w1 = mk(W1l, b1, W1r, g1, be1)
    w2 = mk(W2l, b2, W2r, g2, be2)
    w3 = mk(W3l, b3, W3r, g3, be3)

    degp = _sc_deg(dst_p, ones_c, zst)
    aggp = _sc_agg(src_p, dst_p, x, zst)
    h = _tc_layer(aggp, degp, x, *w1)
    aggp = _sc_agg(src_p, dst_p, h, zst)
    h = _tc_layer(aggp, degp, h, *w2)
    aggp = _sc_agg(src_p, dst_p, h, zst)
    return _tc_layer_head(aggp, degp, h, *w3,
                          Wf.T, bf[None, :], Wc.T, bc[None, :])
